# baseline (speedup 1.0000x reference)
import jax
import jax.numpy as jnp
from jax.experimental import pallas as pl
from jax.experimental.pallas import tpu as pltpu

_CHAR_SIZE = 128
_UNK_IDX = 0
_ROWS_PER_BLOCK = 256
_SUB = 64
_NSUB = _ROWS_PER_BLOCK // _SUB


def _onehot_block(x_ref, o_ref, buf, sems):
    i = pl.program_id(0)
    nsteps = pl.num_programs(0)
    c = x_ref.shape[1]

    for s in range(_NSUB):
        @pl.when(i > 0)
        def _():
            pltpu.make_async_copy(
                buf.at[s],
                o_ref.at[pl.ds((i - 1) * _ROWS_PER_BLOCK + s * _SUB, _SUB)],
                sems.at[s],
            ).wait()

        x = x_ref[pl.ds(i * _ROWS_PER_BLOCK + s * _SUB, _SUB), :]
        lane = jax.lax.broadcasted_iota(
            jnp.int32, (_SUB, c, _CHAR_SIZE), 2
        )
        oh = (x[:, :, None] == lane).astype(jnp.int32)
        if s == _UNK_IDX // _SUB:
            @pl.when(i == _UNK_IDX // _ROWS_PER_BLOCK)
            def _():
                row = jax.lax.broadcasted_iota(
                    jnp.int32, (_SUB, c, _CHAR_SIZE), 0
                )
                buf[s] = jnp.where(row == _UNK_IDX % _SUB, 0, oh)

            @pl.when(i != _UNK_IDX // _ROWS_PER_BLOCK)
            def _():
                buf[s] = oh
        else:
            buf[s] = oh

        pltpu.make_async_copy(
            buf.at[s],
            o_ref.at[pl.ds(i * _ROWS_PER_BLOCK + s * _SUB, _SUB)],
            sems.at[s],
        ).start()

    @pl.when(i == nsteps - 1)
    def _():
        for s in range(_NSUB):
            pltpu.make_async_copy(
                buf.at[s],
                o_ref.at[pl.ds(i * _ROWS_PER_BLOCK + s * _SUB, _SUB)],
                sems.at[s],
            ).wait()


def kernel(x):
    n, c = x.shape
    grid = (n // _ROWS_PER_BLOCK,)
    return pl.pallas_call(
        _onehot_block,
        grid=grid,
        in_specs=[pl.BlockSpec((n, c), lambda i: (0, 0))],
        out_specs=pl.BlockSpec(memory_space=pltpu.MemorySpace.HBM),
        out_shape=jax.ShapeDtypeStruct((n, c, _CHAR_SIZE), jnp.int32),
        scratch_shapes=[
            pltpu.VMEM((_NSUB, _SUB, c, _CHAR_SIZE), jnp.int32),
            pltpu.SemaphoreType.DMA((_NSUB,)),
        ],
    )(x)


# speedup vs baseline: 1.0343x; 1.0343x over previous
import jax
import jax.numpy as jnp
from jax.experimental import pallas as pl
from jax.experimental.pallas import tpu as pltpu

_CHAR_SIZE = 128
_UNK_IDX = 0
_ROWS_PER_BLOCK = 128
_SUB = 32
_NSUB = _ROWS_PER_BLOCK // _SUB


def _onehot_block(x_ref, o_ref, buf, sems):
    i = pl.program_id(0)
    nsteps = pl.num_programs(0)
    c = x_ref.shape[1]

    for s in range(_NSUB):
        @pl.when(i > 0)
        def _():
            pltpu.make_async_copy(
                buf.at[s],
                o_ref.at[pl.ds((i - 1) * _ROWS_PER_BLOCK + s * _SUB, _SUB)],
                sems.at[s],
            ).wait()

        x = x_ref[pl.ds(s * _SUB, _SUB), :]
        lane = jax.lax.broadcasted_iota(
            jnp.int32, (_SUB, c, _CHAR_SIZE), 2
        )
        oh = (x[:, :, None] == lane).astype(jnp.int32)
        if s == _UNK_IDX % _ROWS_PER_BLOCK // _SUB:
            @pl.when(i == _UNK_IDX // _ROWS_PER_BLOCK)
            def _():
                row = jax.lax.broadcasted_iota(
                    jnp.int32, (_SUB, c, _CHAR_SIZE), 0
                )
                buf[s] = jnp.where(row == _UNK_IDX % _SUB, 0, oh)

            @pl.when(i != _UNK_IDX // _ROWS_PER_BLOCK)
            def _():
                buf[s] = oh
        else:
            buf[s] = oh

        pltpu.make_async_copy(
            buf.at[s],
            o_ref.at[pl.ds(i * _ROWS_PER_BLOCK + s * _SUB, _SUB)],
            sems.at[s],
        ).start()

    @pl.when(i == nsteps - 1)
    def _():
        for s in range(_NSUB):
            pltpu.make_async_copy(
                buf.at[s],
                o_ref.at[pl.ds(i * _ROWS_PER_BLOCK + s * _SUB, _SUB)],
                sems.at[s],
            ).wait()


def kernel(x):
    n, c = x.shape
    grid = (n // _ROWS_PER_BLOCK,)
    return pl.pallas_call(
        _onehot_block,
        grid=grid,
        in_specs=[pl.BlockSpec((_ROWS_PER_BLOCK, c), lambda i: (i, 0))],
        out_specs=pl.BlockSpec(memory_space=pltpu.MemorySpace.HBM),
        out_shape=jax.ShapeDtypeStruct((n, c, _CHAR_SIZE), jnp.int32),
        scratch_shapes=[
            pltpu.VMEM((_NSUB, _SUB, c, _CHAR_SIZE), jnp.int32),
            pltpu.SemaphoreType.DMA((_NSUB,)),
        ],
    )(x)
